# Initial kernel scaffold; baseline (speedup 1.0000x reference)
import jax
import jax.numpy as jnp
from jax.experimental import pallas as pl


def kernel(Y, edge_index, edge_values):
    raise NotImplementedError("write your pallas kernel here")


# speedup vs baseline: 54.3946x; 54.3946x over previous
import jax
import jax.numpy as jnp
from jax import lax
from jax.experimental import pallas as pl
from jax.experimental.pallas import tpu as pltpu
from jax.experimental.pallas import tpu_sc as plsc

_N = 100000
_G = 16
_E = 3200000

_NC = 2
_NS = 16
_NW = _NC * _NS
_EPW = _E // _NW
_B = 2000
_NBLK = _EPW // _B


def _sc_body(ei_hbm, y_hbm, out_hbm, idx_s, idx_d, rows_s, rows_d, acc_v,
             sem_s, sem_d):
    c = lax.axis_index("c")
    s = lax.axis_index("s")
    wid = s * _NC + c
    base = wid * _EPW

    def blk(b, acc):
        off = base + b * _B
        pltpu.sync_copy(ei_hbm.at[pl.ds(off, _B)], idx_s)
        pltpu.sync_copy(ei_hbm.at[pl.ds(_E + off, _B)], idx_d)
        cp_s = pltpu.async_copy(y_hbm.at[idx_s], rows_s, sem_s)
        cp_d = pltpu.async_copy(y_hbm.at[idx_d], rows_d, sem_d)
        cp_s.wait()
        cp_d.wait()

        def edge(e, a):
            return a + rows_s[e, :] * (1.0 - rows_d[e, :])

        return acc + lax.fori_loop(0, _B, edge, jnp.zeros((_G,), jnp.float32),
                                   unroll=8)

    acc = lax.fori_loop(0, _NBLK, blk, jnp.zeros((_G,), jnp.float32))
    acc_v[...] = acc
    pltpu.sync_copy(acc_v, out_hbm.at[wid])


_sc_loss1 = pl.kernel(
    _sc_body,
    out_type=jax.ShapeDtypeStruct((_NW, _G), jnp.float32),
    mesh=plsc.VectorSubcoreMesh(core_axis_name="c", subcore_axis_name="s",
                                num_cores=_NC, num_subcores=_NS),
    scratch_types=[
        pltpu.VMEM((_B,), jnp.int32),
        pltpu.VMEM((_B,), jnp.int32),
        pltpu.VMEM((_B, _G), jnp.float32),
        pltpu.VMEM((_B, _G), jnp.float32),
        pltpu.VMEM((_G,), jnp.float32),
        pltpu.SemaphoreType.DMA,
        pltpu.SemaphoreType.DMA,
    ],
    compiler_params=pltpu.CompilerParams(use_tc_tiling_on_sc=False),
)

_KT = 100
_EVR, _EVC = 800, 4000
_YB = _N // _KT


def _tc_body(ev_ref, y_ref, gam_ref, l2_ref, gacc_ref, cacc_ref):
    i = pl.program_id(0)

    @pl.when(i == 0)
    def _():
        gacc_ref[0] = 0.0
        cacc_ref[...] = jnp.zeros_like(cacc_ref)

    gacc_ref[0] += jnp.sum(ev_ref[...])
    cacc_ref[...] += jnp.sum(y_ref[...], axis=0, keepdims=True)

    @pl.when(i == _KT - 1)
    def _():
        gam_ref[0, 0] = gacc_ref[0]
        d = cacc_ref[...] - (jnp.float32(_N) / jnp.float32(_G))
        l2_ref[0, 0] = jnp.sum(d * d)


_tc_stats = pl.pallas_call(
    _tc_body,
    grid=(_KT,),
    in_specs=[
        pl.BlockSpec((_EVR // _KT, _EVC), lambda i: (i, 0)),
        pl.BlockSpec((_YB, _G), lambda i: (i, 0)),
    ],
    out_specs=[
        pl.BlockSpec(memory_space=pltpu.SMEM),
        pl.BlockSpec(memory_space=pltpu.SMEM),
    ],
    out_shape=[
        jax.ShapeDtypeStruct((1, 1), jnp.float32),
        jax.ShapeDtypeStruct((1, 1), jnp.float32),
    ],
    scratch_shapes=[
        pltpu.SMEM((1,), jnp.float32),
        pltpu.VMEM((1, _G), jnp.float32),
    ],
)


def kernel(Y, edge_index, edge_values):
    partials = _sc_loss1(edge_index.reshape(-1), Y)
    gamma, l2 = _tc_stats(edge_values.reshape(_EVR, _EVC), Y)
    loss_1 = (jnp.sum(partials) / gamma[0, 0]).reshape(1)
    loss_2 = l2.reshape(1)
    loss = loss_1 + loss_2
    return (loss, loss_1, loss_2, Y)


# speedup vs baseline: 74.2958x; 1.3659x over previous
import jax
import jax.numpy as jnp
from jax import lax
from jax.experimental import pallas as pl
from jax.experimental.pallas import tpu as pltpu
from jax.experimental.pallas import tpu_sc as plsc

_N = 100000
_G = 16
_E = 3200000

_NC = 2
_NS = 16
_NW = _NC * _NS
_EPW = _E // _NW
_B = 1000
_NBLK = _EPW // _B
_NP = _NBLK // 2


def _sc_body(ei_hbm, y_hbm, out_hbm,
             idx_s0, idx_d0, idx_s1, idx_d1,
             rows_s0, rows_d0, rows_s1, rows_d1,
             acc_v, sem_g0, sem_g1, sem_i0, sem_i1):
    c = lax.axis_index("c")
    s = lax.axis_index("s")
    wid = s * _NC + c
    base = wid * _EPW

    slots = ((idx_s0, idx_d0, rows_s0, rows_d0, sem_g0, sem_i0),
             (idx_s1, idx_d1, rows_s1, rows_d1, sem_g1, sem_i1))

    def idx_start(b, slot):
        i_s, i_d, _, _, _, sem_i = slots[slot]
        off = base + b * _B
        pltpu.async_copy(ei_hbm.at[pl.ds(off, _B)], i_s, sem_i)
        pltpu.async_copy(ei_hbm.at[pl.ds(_E + off, _B)], i_d, sem_i)

    def idx_wait(slot):
        i_s, i_d, _, _, _, sem_i = slots[slot]
        pltpu.make_async_copy(ei_hbm.at[pl.ds(0, _B)], i_s, sem_i).wait()
        pltpu.make_async_copy(ei_hbm.at[pl.ds(0, _B)], i_d, sem_i).wait()

    def gather_start(slot):
        i_s, i_d, r_s, r_d, sem_g, _ = slots[slot]
        pltpu.async_copy(y_hbm.at[i_s], r_s, sem_g)
        pltpu.async_copy(y_hbm.at[i_d], r_d, sem_g)

    def gather_wait(slot):
        i_s, i_d, r_s, r_d, sem_g, _ = slots[slot]
        pltpu.make_async_copy(y_hbm.at[i_s], r_s, sem_g).wait()
        pltpu.make_async_copy(y_hbm.at[i_d], r_d, sem_g).wait()

    def compute(slot, acc):
        _, _, r_s, r_d, _, _ = slots[slot]

        def edge(e, a):
            return a + r_s[e, :] * (1.0 - r_d[e, :])

        return acc + lax.fori_loop(0, _B, edge, jnp.zeros((_G,), jnp.float32),
                                   unroll=8)

    idx_start(0, 0)
    idx_wait(0)
    gather_start(0)
    idx_start(1, 1)

    def pair(p, acc):
        b0 = 2 * p
        idx_wait(1)
        gather_start(1)
        gather_wait(0)

        @pl.when(p < _NP - 1)
        def _():
            idx_start(b0 + 2, 0)

        acc = compute(0, acc)
        gather_wait(1)

        @pl.when(p < _NP - 1)
        def _():
            idx_wait(0)
            gather_start(0)
            idx_start(b0 + 3, 1)

        acc = compute(1, acc)
        return acc

    acc = lax.fori_loop(0, _NP, pair, jnp.zeros((_G,), jnp.float32))
    acc_v[...] = acc
    pltpu.sync_copy(acc_v, out_hbm.at[wid])


_sc_loss1 = pl.kernel(
    _sc_body,
    out_type=jax.ShapeDtypeStruct((_NW, _G), jnp.float32),
    mesh=plsc.VectorSubcoreMesh(core_axis_name="c", subcore_axis_name="s",
                                num_cores=_NC, num_subcores=_NS),
    scratch_types=[
        pltpu.VMEM((_B,), jnp.int32),
        pltpu.VMEM((_B,), jnp.int32),
        pltpu.VMEM((_B,), jnp.int32),
        pltpu.VMEM((_B,), jnp.int32),
        pltpu.VMEM((_B, _G), jnp.float32),
        pltpu.VMEM((_B, _G), jnp.float32),
        pltpu.VMEM((_B, _G), jnp.float32),
        pltpu.VMEM((_B, _G), jnp.float32),
        pltpu.VMEM((_G,), jnp.float32),
        pltpu.SemaphoreType.DMA,
        pltpu.SemaphoreType.DMA,
        pltpu.SemaphoreType.DMA,
        pltpu.SemaphoreType.DMA,
    ],
    compiler_params=pltpu.CompilerParams(use_tc_tiling_on_sc=False),
)

_KT = 100
_EVR, _EVC = 800, 4000
_YB = _N // _KT


def _tc_body(ev_ref, y_ref, gam_ref, l2_ref, gacc_ref, cacc_ref):
    i = pl.program_id(0)

    @pl.when(i == 0)
    def _():
        gacc_ref[0] = 0.0
        cacc_ref[...] = jnp.zeros_like(cacc_ref)

    gacc_ref[0] += jnp.sum(ev_ref[...])
    cacc_ref[...] += jnp.sum(y_ref[...], axis=0, keepdims=True)

    @pl.when(i == _KT - 1)
    def _():
        gam_ref[0, 0] = gacc_ref[0]
        d = cacc_ref[...] - (jnp.float32(_N) / jnp.float32(_G))
        l2_ref[0, 0] = jnp.sum(d * d)


_tc_stats = pl.pallas_call(
    _tc_body,
    grid=(_KT,),
    in_specs=[
        pl.BlockSpec((_EVR // _KT, _EVC), lambda i: (i, 0)),
        pl.BlockSpec((_YB, _G), lambda i: (i, 0)),
    ],
    out_specs=[
        pl.BlockSpec(memory_space=pltpu.SMEM),
        pl.BlockSpec(memory_space=pltpu.SMEM),
    ],
    out_shape=[
        jax.ShapeDtypeStruct((1, 1), jnp.float32),
        jax.ShapeDtypeStruct((1, 1), jnp.float32),
    ],
    scratch_shapes=[
        pltpu.SMEM((1,), jnp.float32),
        pltpu.VMEM((1, _G), jnp.float32),
    ],
)


def kernel(Y, edge_index, edge_values):
    partials = _sc_loss1(edge_index.reshape(-1), Y)
    gamma, l2 = _tc_stats(edge_values.reshape(_EVR, _EVC), Y)
    loss_1 = (jnp.sum(partials) / gamma[0, 0]).reshape(1)
    loss_2 = l2.reshape(1)
    loss = loss_1 + loss_2
    return (loss, loss_1, loss_2, Y)


# speedup vs baseline: 74.4051x; 1.0015x over previous
import jax
import jax.numpy as jnp
from jax import lax
from jax.experimental import pallas as pl
from jax.experimental.pallas import tpu as pltpu
from jax.experimental.pallas import tpu_sc as plsc

_N = 100000
_G = 16
_E = 3200000

_NC = 2
_NS = 16
_NW = _NC * _NS
_EPW = _E // _NW
_B = 1000
_NBLK = _EPW // _B
_NP = _NBLK // 2


def _sc_body(ei_hbm, y_hbm, out_hbm,
             idx_s0, idx_d0, idx_s1, idx_d1,
             rows_s0, rows_d0, rows_s1, rows_d1,
             acc_v, sem_g0, sem_g1, sem_i0, sem_i1):
    c = lax.axis_index("c")
    s = lax.axis_index("s")
    wid = s * _NC + c
    base = wid * _EPW

    slots = ((idx_s0, idx_d0, rows_s0, rows_d0, sem_g0, sem_i0),
             (idx_s1, idx_d1, rows_s1, rows_d1, sem_g1, sem_i1))

    def idx_start(b, slot):
        i_s, i_d, _, _, _, sem_i = slots[slot]
        off = base + b * _B
        pltpu.async_copy(ei_hbm.at[pl.ds(off, _B)], i_s, sem_i)
        pltpu.async_copy(ei_hbm.at[pl.ds(_E + off, _B)], i_d, sem_i)

    def idx_wait(slot):
        i_s, i_d, _, _, _, sem_i = slots[slot]
        pltpu.make_async_copy(ei_hbm.at[pl.ds(0, _B)], i_s, sem_i).wait()
        pltpu.make_async_copy(ei_hbm.at[pl.ds(0, _B)], i_d, sem_i).wait()

    def gather_start(slot):
        i_s, i_d, r_s, r_d, sem_g, _ = slots[slot]
        pltpu.async_copy(y_hbm.at[i_s], r_s, sem_g)
        pltpu.async_copy(y_hbm.at[i_d], r_d, sem_g)

    def gather_wait(slot):
        i_s, i_d, r_s, r_d, sem_g, _ = slots[slot]
        pltpu.make_async_copy(y_hbm.at[i_s], r_s, sem_g).wait()
        pltpu.make_async_copy(y_hbm.at[i_d], r_d, sem_g).wait()

    def compute(slot, acc):
        _, _, r_s, r_d, _, _ = slots[slot]
        u = 8

        def step(i, accs):
            e = i * u
            return tuple(
                a + r_s[e + j, :] * (1.0 - r_d[e + j, :])
                for j, a in enumerate(accs)
            )

        accs = lax.fori_loop(
            0, _B // u, step,
            tuple(jnp.zeros((_G,), jnp.float32) for _ in range(u)))
        block = accs[0]
        for a in accs[1:]:
            block = block + a
        return acc + block

    idx_start(0, 0)
    idx_wait(0)
    gather_start(0)
    idx_start(1, 1)

    def pair(p, acc):
        b0 = 2 * p
        idx_wait(1)
        gather_start(1)
        gather_wait(0)

        @pl.when(p < _NP - 1)
        def _():
            idx_start(b0 + 2, 0)

        acc = compute(0, acc)
        gather_wait(1)

        @pl.when(p < _NP - 1)
        def _():
            idx_wait(0)
            gather_start(0)
            idx_start(b0 + 3, 1)

        acc = compute(1, acc)
        return acc

    acc = lax.fori_loop(0, _NP, pair, jnp.zeros((_G,), jnp.float32))
    acc_v[...] = acc
    pltpu.sync_copy(acc_v, out_hbm.at[wid])


_sc_loss1 = pl.kernel(
    _sc_body,
    out_type=jax.ShapeDtypeStruct((_NW, _G), jnp.float32),
    mesh=plsc.VectorSubcoreMesh(core_axis_name="c", subcore_axis_name="s",
                                num_cores=_NC, num_subcores=_NS),
    scratch_types=[
        pltpu.VMEM((_B,), jnp.int32),
        pltpu.VMEM((_B,), jnp.int32),
        pltpu.VMEM((_B,), jnp.int32),
        pltpu.VMEM((_B,), jnp.int32),
        pltpu.VMEM((_B, _G), jnp.float32),
        pltpu.VMEM((_B, _G), jnp.float32),
        pltpu.VMEM((_B, _G), jnp.float32),
        pltpu.VMEM((_B, _G), jnp.float32),
        pltpu.VMEM((_G,), jnp.float32),
        pltpu.SemaphoreType.DMA,
        pltpu.SemaphoreType.DMA,
        pltpu.SemaphoreType.DMA,
        pltpu.SemaphoreType.DMA,
    ],
    compiler_params=pltpu.CompilerParams(use_tc_tiling_on_sc=False),
)

_KT = 100
_EVR, _EVC = 800, 4000
_YB = _N // _KT


def _tc_body(ev_ref, y_ref, gam_ref, l2_ref, gacc_ref, cacc_ref):
    i = pl.program_id(0)

    @pl.when(i == 0)
    def _():
        gacc_ref[0] = 0.0
        cacc_ref[...] = jnp.zeros_like(cacc_ref)

    gacc_ref[0] += jnp.sum(ev_ref[...])
    cacc_ref[...] += jnp.sum(y_ref[...], axis=0, keepdims=True)

    @pl.when(i == _KT - 1)
    def _():
        gam_ref[0, 0] = gacc_ref[0]
        d = cacc_ref[...] - (jnp.float32(_N) / jnp.float32(_G))
        l2_ref[0, 0] = jnp.sum(d * d)


_tc_stats = pl.pallas_call(
    _tc_body,
    grid=(_KT,),
    in_specs=[
        pl.BlockSpec((_EVR // _KT, _EVC), lambda i: (i, 0)),
        pl.BlockSpec((_YB, _G), lambda i: (i, 0)),
    ],
    out_specs=[
        pl.BlockSpec(memory_space=pltpu.SMEM),
        pl.BlockSpec(memory_space=pltpu.SMEM),
    ],
    out_shape=[
        jax.ShapeDtypeStruct((1, 1), jnp.float32),
        jax.ShapeDtypeStruct((1, 1), jnp.float32),
    ],
    scratch_shapes=[
        pltpu.SMEM((1,), jnp.float32),
        pltpu.VMEM((1, _G), jnp.float32),
    ],
)


def kernel(Y, edge_index, edge_values):
    partials = _sc_loss1(edge_index.reshape(-1), Y)
    gamma, l2 = _tc_stats(edge_values.reshape(_EVR, _EVC), Y)
    loss_1 = (jnp.sum(partials) / gamma[0, 0]).reshape(1)
    loss_2 = l2.reshape(1)
    loss = loss_1 + loss_2
    return (loss, loss_1, loss_2, Y)


# speedup vs baseline: 97.3768x; 1.3087x over previous
import jax
import jax.numpy as jnp
from jax import lax
from jax.experimental import pallas as pl
from jax.experimental.pallas import tpu as pltpu
from jax.experimental.pallas import tpu_sc as plsc

_N = 100000
_G = 16
_E = 3200000

_NC = 2
_NS = 16
_NW = _NC * _NS
_EPW = _E // _NW
_B = 400
_NBLK = _EPW // _B
_NP = _NBLK // 2


def _sc_body(ei_hbm, y_hbm, out_hbm,
             idx_s0, idx_d0, idx_s1, idx_d1,
             rows_s0, rows_d0, rows_s1, rows_d1,
             acc_v, y_sp, sem_g0, sem_g1, sem_i0, sem_i1):
    c = lax.axis_index("c")
    s = lax.axis_index("s")
    wid = s * _NC + c
    base = wid * _EPW

    slots = ((idx_s0, idx_d0, rows_s0, rows_d0, sem_g0, sem_i0),
             (idx_s1, idx_d1, rows_s1, rows_d1, sem_g1, sem_i1))

    def idx_start(b, slot):
        i_s, i_d, _, _, _, sem_i = slots[slot]
        off = base + b * _B
        pltpu.async_copy(ei_hbm.at[pl.ds(off, _B)], i_s, sem_i)
        pltpu.async_copy(ei_hbm.at[pl.ds(_E + off, _B)], i_d, sem_i)

    def idx_wait(slot):
        i_s, i_d, _, _, _, sem_i = slots[slot]
        pltpu.make_async_copy(ei_hbm.at[pl.ds(0, _B)], i_s, sem_i).wait()
        pltpu.make_async_copy(ei_hbm.at[pl.ds(0, _B)], i_d, sem_i).wait()

    def gather_start(slot):
        i_s, i_d, r_s, r_d, sem_g, _ = slots[slot]
        pltpu.async_copy(y_sp.at[i_s], r_s, sem_g)
        pltpu.async_copy(y_sp.at[i_d], r_d, sem_g)

    def gather_wait(slot):
        i_s, i_d, r_s, r_d, sem_g, _ = slots[slot]
        pltpu.make_async_copy(y_sp.at[i_s], r_s, sem_g).wait()
        pltpu.make_async_copy(y_sp.at[i_d], r_d, sem_g).wait()

    def compute(slot, acc):
        _, _, r_s, r_d, _, _ = slots[slot]
        u = 8

        def step(i, accs):
            e = i * u
            return tuple(
                a + r_s[e + j, :] * (1.0 - r_d[e + j, :])
                for j, a in enumerate(accs)
            )

        accs = lax.fori_loop(
            0, _B // u, step,
            tuple(jnp.zeros((_G,), jnp.float32) for _ in range(u)))
        block = accs[0]
        for a in accs[1:]:
            block = block + a
        return acc + block

    idx_start(0, 0)
    idx_start(1, 1)
    rpt = _N // _NS
    roff = s * rpt
    pltpu.sync_copy(y_hbm.at[pl.ds(roff, rpt)], y_sp.at[pl.ds(roff, rpt)])
    plsc.subcore_barrier()

    idx_wait(0)
    gather_start(0)

    def pair(p, acc):
        b0 = 2 * p
        idx_wait(1)
        gather_start(1)
        gather_wait(0)

        @pl.when(p < _NP - 1)
        def _():
            idx_start(b0 + 2, 0)

        acc = compute(0, acc)
        gather_wait(1)

        @pl.when(p < _NP - 1)
        def _():
            idx_wait(0)
            gather_start(0)
            idx_start(b0 + 3, 1)

        acc = compute(1, acc)
        return acc

    acc = lax.fori_loop(0, _NP, pair, jnp.zeros((_G,), jnp.float32))
    acc_v[...] = acc
    pltpu.sync_copy(acc_v, out_hbm.at[wid])


_sc_loss1 = pl.kernel(
    _sc_body,
    out_type=jax.ShapeDtypeStruct((_NW, _G), jnp.float32),
    mesh=plsc.VectorSubcoreMesh(core_axis_name="c", subcore_axis_name="s",
                                num_cores=_NC, num_subcores=_NS),
    scratch_types=[
        pltpu.VMEM((_B,), jnp.int32),
        pltpu.VMEM((_B,), jnp.int32),
        pltpu.VMEM((_B,), jnp.int32),
        pltpu.VMEM((_B,), jnp.int32),
        pltpu.VMEM((_B, _G), jnp.float32),
        pltpu.VMEM((_B, _G), jnp.float32),
        pltpu.VMEM((_B, _G), jnp.float32),
        pltpu.VMEM((_B, _G), jnp.float32),
        pltpu.VMEM((_G,), jnp.float32),
        pltpu.VMEM_SHARED((_N, _G), jnp.float32),
        pltpu.SemaphoreType.DMA,
        pltpu.SemaphoreType.DMA,
        pltpu.SemaphoreType.DMA,
        pltpu.SemaphoreType.DMA,
    ],
    compiler_params=pltpu.CompilerParams(use_tc_tiling_on_sc=False),
)

_KT = 100
_EVR, _EVC = 800, 4000
_YB = _N // _KT


def _tc_body(ev_ref, y_ref, gam_ref, l2_ref, gacc_ref, cacc_ref):
    i = pl.program_id(0)

    @pl.when(i == 0)
    def _():
        gacc_ref[0] = 0.0
        cacc_ref[...] = jnp.zeros_like(cacc_ref)

    gacc_ref[0] += jnp.sum(ev_ref[...])
    cacc_ref[...] += jnp.sum(y_ref[...], axis=0, keepdims=True)

    @pl.when(i == _KT - 1)
    def _():
        gam_ref[0, 0] = gacc_ref[0]
        d = cacc_ref[...] - (jnp.float32(_N) / jnp.float32(_G))
        l2_ref[0, 0] = jnp.sum(d * d)


_tc_stats = pl.pallas_call(
    _tc_body,
    grid=(_KT,),
    in_specs=[
        pl.BlockSpec((_EVR // _KT, _EVC), lambda i: (i, 0)),
        pl.BlockSpec((_YB, _G), lambda i: (i, 0)),
    ],
    out_specs=[
        pl.BlockSpec(memory_space=pltpu.SMEM),
        pl.BlockSpec(memory_space=pltpu.SMEM),
    ],
    out_shape=[
        jax.ShapeDtypeStruct((1, 1), jnp.float32),
        jax.ShapeDtypeStruct((1, 1), jnp.float32),
    ],
    scratch_shapes=[
        pltpu.SMEM((1,), jnp.float32),
        pltpu.VMEM((1, _G), jnp.float32),
    ],
)


def kernel(Y, edge_index, edge_values):
    partials = _sc_loss1(edge_index.reshape(-1), Y)
    gamma, l2 = _tc_stats(edge_values.reshape(_EVR, _EVC), Y)
    loss_1 = (jnp.sum(partials) / gamma[0, 0]).reshape(1)
    loss_2 = l2.reshape(1)
    loss = loss_1 + loss_2
    return (loss, loss_1, loss_2, Y)
